# Initial kernel scaffold; baseline (speedup 1.0000x reference)
import functools

import jax
import jax.numpy as jnp
from jax import lax
from jax.experimental import pallas as pl
from jax.experimental.pallas import tpu as pltpu
from jax.experimental.pallas import tpu_sc as plsc

NC = 2
NS = 16
NW = NC * NS

LANES = 16

ROWS_PER_GROUP = 400
BATCH_PER_GROUP = 8
STREAM_CHUNKS = (104, 104, 104, 88)


def _sc_pool(x2, emb, batch, hist, embed):
    rows_per_worker = x2.shape[1]
    batch_per_worker = batch // NW
    groups = rows_per_worker // ROWS_PER_GROUP
    assert rows_per_worker == batch_per_worker * hist
    assert rows_per_worker % ROWS_PER_GROUP == 0
    assert ROWS_PER_GROUP == BATCH_PER_GROUP * hist
    assert embed % LANES == 0
    ecols = embed // LANES
    assert groups % 2 == 0

    mesh = plsc.VectorSubcoreMesh(
        core_axis_name="c", subcore_axis_name="s",
        num_cores=NC, num_subcores=NS)

    @functools.partial(
        pl.kernel,
        out_type=jax.ShapeDtypeStruct((batch, embed), jnp.float32),
        mesh=mesh,
        scratch_types=[
            pltpu.VMEM((rows_per_worker,), jnp.int32),
            pltpu.VMEM((ROWS_PER_GROUP, embed), jnp.float32),
            pltpu.VMEM((ROWS_PER_GROUP, embed), jnp.float32),
            pltpu.VMEM((batch_per_worker, embed), jnp.float32),
            pltpu.SemaphoreType.DMA,
            pltpu.SemaphoreType.DMA,
        ],
    )
    def k(x_hbm, emb_hbm, out_hbm, idx_v, buf0, buf1, pooled, sem0, sem1):
        wid = lax.axis_index("s") * NC + lax.axis_index("c")

        pltpu.sync_copy(x_hbm.at[wid], idx_v)

        def fire(g, buf, sem):
            base = g * ROWS_PER_GROUP
            off = 0
            for n in STREAM_CHUNKS:
                src_idx = idx_v.at[pl.ds(pl.multiple_of(base + off, 8), n)]
                pltpu.async_copy(emb_hbm.at[src_idx],
                                 buf.at[pl.ds(off, n)], sem)
                off += n

        def drain(buf, sem):
            pltpu.make_async_copy(
                emb_hbm.at[pl.ds(0, ROWS_PER_GROUP)], buf, sem).wait()

        def compute(buf, g):
            for bb in range(BATCH_PER_GROUP):
                fb = bb * hist

                def body(l, accs, fb=fb):
                    out = list(accs)
                    for kk in range(10):
                        row = fb + l * 10 + kk
                        for c in range(ecols):
                            out[c] = out[c] + buf[row, pl.ds(c * LANES, LANES)]
                    return tuple(out)

                zero = tuple(jnp.zeros((LANES,), jnp.float32)
                             for _ in range(ecols))
                accs = lax.fori_loop(0, hist // 10, body, zero)
                prow = g * BATCH_PER_GROUP + bb
                for c in range(ecols):
                    pooled[prow, pl.ds(c * LANES, LANES)] = accs[c]

        fire(0, buf0, sem0)

        def loop_body(i, carry):
            fire(2 * i + 1, buf1, sem1)
            drain(buf0, sem0)
            compute(buf0, 2 * i)

            @pl.when(i < groups // 2 - 1)
            def _():
                fire(2 * i + 2, buf0, sem0)

            drain(buf1, sem1)
            compute(buf1, 2 * i + 1)
            return carry

        lax.fori_loop(0, groups // 2, loop_body, 0)

        pltpu.sync_copy(
            pooled, out_hbm.at[pl.ds(wid * batch_per_worker, batch_per_worker)])

    return k(x2, emb)


def _tc_project(pooled, W, b2, hist):
    inv = 1.0 / float(hist)

    def body(h_ref, w_ref, b_ref, o_ref):
        w = w_ref[...] * inv
        o_ref[...] = (
            jnp.dot(h_ref[...], w, preferred_element_type=jnp.float32)
            + b_ref[...])

    return pl.pallas_call(
        body,
        out_shape=jax.ShapeDtypeStruct((pooled.shape[0], W.shape[1]),
                                       jnp.float32),
    )(pooled, W, b2)


def kernel(x, emb, W, b):
    batch, hist = x.shape
    embed = emb.shape[1]
    x2 = x.astype(jnp.int32).reshape(NW, (batch // NW) * hist)
    pooled = _sc_pool(x2, emb, batch, hist, embed)
    return _tc_project(pooled, W, b.reshape(1, -1), hist)


# speedup vs baseline: 2.7689x; 2.7689x over previous
import functools

import jax
import jax.numpy as jnp
from jax import lax
from jax.experimental import pallas as pl
from jax.experimental.pallas import tpu as pltpu
from jax.experimental.pallas import tpu_sc as plsc

NC = 2
NS = 16
NW = NC * NS

LANES = 16

ROWS_PER_GROUP = 400
BATCH_PER_GROUP = 8
STREAM_CHUNKS = (104, 104, 104, 88)


def _sc_pool(x2, emb, batch, hist, embed):
    rows_per_worker = x2.shape[1]
    batch_per_worker = batch // NW
    groups = rows_per_worker // ROWS_PER_GROUP
    assert rows_per_worker == batch_per_worker * hist
    assert rows_per_worker % ROWS_PER_GROUP == 0
    assert ROWS_PER_GROUP == BATCH_PER_GROUP * hist
    assert embed % LANES == 0
    ecols = embed // LANES
    assert groups % 2 == 0

    mesh = plsc.VectorSubcoreMesh(
        core_axis_name="c", subcore_axis_name="s",
        num_cores=NC, num_subcores=NS)

    @functools.partial(
        pl.kernel,
        out_type=jax.ShapeDtypeStruct((batch, embed), jnp.float32),
        mesh=mesh,
        scratch_types=[
            pltpu.VMEM((rows_per_worker,), jnp.int32),
            pltpu.VMEM((ROWS_PER_GROUP, embed), jnp.float32),
            pltpu.VMEM((ROWS_PER_GROUP, embed), jnp.float32),
            pltpu.VMEM((batch_per_worker, embed), jnp.float32),
            pltpu.SemaphoreType.DMA,
            pltpu.SemaphoreType.DMA,
        ],
        compiler_params=pltpu.CompilerParams(use_tc_tiling_on_sc=False),
    )
    def k(x_hbm, emb_hbm, out_hbm, idx_v, buf0, buf1, pooled, sem0, sem1):
        wid = lax.axis_index("s") * NC + lax.axis_index("c")

        pltpu.sync_copy(x_hbm.at[wid], idx_v)

        def fire(g, buf, sem):
            base = g * ROWS_PER_GROUP
            off = 0
            for n in STREAM_CHUNKS:
                src_idx = idx_v.at[pl.ds(pl.multiple_of(base + off, 8), n)]
                pltpu.async_copy(emb_hbm.at[src_idx],
                                 buf.at[pl.ds(off, n)], sem)
                off += n

        def drain(buf, sem):
            pltpu.make_async_copy(
                emb_hbm.at[pl.ds(0, ROWS_PER_GROUP)], buf, sem).wait()

        def compute(buf, g):
            for bb in range(BATCH_PER_GROUP):
                fb = bb * hist

                def body(l, accs, fb=fb):
                    out = list(accs)
                    for kk in range(10):
                        row = fb + l * 10 + kk
                        for c in range(ecols):
                            out[c] = out[c] + buf[row, pl.ds(c * LANES, LANES)]
                    return tuple(out)

                zero = tuple(jnp.zeros((LANES,), jnp.float32)
                             for _ in range(ecols))
                accs = lax.fori_loop(0, hist // 10, body, zero)
                prow = g * BATCH_PER_GROUP + bb
                for c in range(ecols):
                    pooled[prow, pl.ds(c * LANES, LANES)] = accs[c]

        fire(0, buf0, sem0)

        def loop_body(i, carry):
            fire(2 * i + 1, buf1, sem1)
            drain(buf0, sem0)
            compute(buf0, 2 * i)

            @pl.when(i < groups // 2 - 1)
            def _():
                fire(2 * i + 2, buf0, sem0)

            drain(buf1, sem1)
            compute(buf1, 2 * i + 1)
            return carry

        lax.fori_loop(0, groups // 2, loop_body, 0)

        pltpu.sync_copy(
            pooled, out_hbm.at[pl.ds(wid * batch_per_worker, batch_per_worker)])

    return k(x2, emb)


def _tc_project(pooled, W, b2, hist):
    inv = 1.0 / float(hist)

    def body(h_ref, w_ref, b_ref, o_ref):
        w = w_ref[...] * inv
        o_ref[...] = (
            jnp.dot(h_ref[...], w, preferred_element_type=jnp.float32)
            + b_ref[...])

    return pl.pallas_call(
        body,
        out_shape=jax.ShapeDtypeStruct((pooled.shape[0], W.shape[1]),
                                       jnp.float32),
    )(pooled, W, b2)


def kernel(x, emb, W, b):
    batch, hist = x.shape
    embed = emb.shape[1]
    x2 = x.astype(jnp.int32).reshape(NW, (batch // NW) * hist)
    pooled = _sc_pool(x2, emb, batch, hist, embed)
    return _tc_project(pooled, W, b.reshape(1, -1), hist)


# speedup vs baseline: 9.2519x; 3.3413x over previous
import functools

import jax
import jax.numpy as jnp
from jax import lax
from jax.experimental import pallas as pl
from jax.experimental.pallas import tpu as pltpu
from jax.experimental.pallas import tpu_sc as plsc

NC = 2
NS = 16
NW = NC * NS

LANES = 16

VCHUNK = 16384

IDX_PER_STREAM = 128
STREAMS_PER_WAVE = 8
ROWS_PER_WAVE = IDX_PER_STREAM * STREAMS_PER_WAVE


def _tc_project(embT, Wt, b2, hist):
    E, V = embT.shape
    inv = 1.0 / float(hist)
    grid = (V + VCHUNK - 1) // VCHUNK

    def body(embT_ref, wt_ref, b_ref, o0_ref, o1_ref):
        p = jnp.dot(wt_ref[...], embT_ref[...],
                    preferred_element_type=jnp.float32)
        p = (p + b_ref[...]) * inv
        o0_ref[...] = p[0]
        o1_ref[...] = p[1]

    return pl.pallas_call(
        body,
        grid=(grid,),
        in_specs=[
            pl.BlockSpec((E, VCHUNK), lambda i: (0, i)),
            pl.BlockSpec((2, E), lambda i: (0, 0)),
            pl.BlockSpec((2, 1), lambda i: (0, 0)),
        ],
        out_specs=[
            pl.BlockSpec((VCHUNK,), lambda i: (i,)),
            pl.BlockSpec((VCHUNK,), lambda i: (i,)),
        ],
        out_shape=[
            jax.ShapeDtypeStruct((V,), jnp.float32),
            jax.ShapeDtypeStruct((V,), jnp.float32),
        ],
    )(embT, Wt, b2)


def _sc_gather_pool(x2, p0, p1, batch, hist):
    rows_per_worker = x2.shape[1]
    batch_per_worker = batch // NW
    waves = rows_per_worker // ROWS_PER_WAVE
    pool_groups = batch_per_worker // LANES
    assert rows_per_worker == batch_per_worker * hist
    assert rows_per_worker % ROWS_PER_WAVE == 0
    assert batch_per_worker % LANES == 0

    mesh = plsc.VectorSubcoreMesh(
        core_axis_name="c", subcore_axis_name="s",
        num_cores=NC, num_subcores=NS)

    @functools.partial(
        pl.kernel,
        out_type=[
            jax.ShapeDtypeStruct((batch,), jnp.float32),
            jax.ShapeDtypeStruct((batch,), jnp.float32),
        ],
        mesh=mesh,
        scratch_types=[
            pltpu.VMEM((rows_per_worker,), jnp.int32),
            pltpu.VMEM((rows_per_worker,), jnp.float32),
            pltpu.VMEM((rows_per_worker,), jnp.float32),
            pltpu.VMEM((batch_per_worker,), jnp.float32),
            pltpu.VMEM((batch_per_worker,), jnp.float32),
            pltpu.SemaphoreType.DMA,
        ],
    )
    def k(x_hbm, p0_hbm, p1_hbm, out0_hbm, out1_hbm,
          idx_v, b0, b1, ob0, ob1, sem):
        wid = lax.axis_index("s") * NC + lax.axis_index("c")

        pltpu.sync_copy(x_hbm.at[wid], idx_v)

        def fire(g):
            base = g * ROWS_PER_WAVE
            for t in range(STREAMS_PER_WAVE):
                off = pl.multiple_of(base + t * IDX_PER_STREAM, 8)
                sl = pl.ds(off, IDX_PER_STREAM)
                pltpu.async_copy(p0_hbm.at[idx_v.at[sl]], b0.at[sl], sem)
                pltpu.async_copy(p1_hbm.at[idx_v.at[sl]], b1.at[sl], sem)

        def drain_wave():
            sl = pl.ds(0, ROWS_PER_WAVE)
            pltpu.make_async_copy(p0_hbm.at[sl], b0.at[sl], sem).wait()
            pltpu.make_async_copy(p1_hbm.at[sl], b1.at[sl], sem).wait()

        fire(0)

        def gather_body(g, carry):
            fire(g)
            drain_wave()
            return carry

        lax.fori_loop(1, waves, gather_body, 0)
        drain_wave()

        def pool_body(g, carry):
            base = g * LANES * hist

            def inner(l, accs):
                a0, a1 = accs
                sl = pl.ds(base + l * LANES, LANES)
                a0 = a0 + b0[sl]
                a1 = a1 + b1[sl]
                return (a0, a1)

            zero = jnp.zeros((LANES,), jnp.float32)
            a0, a1 = lax.fori_loop(0, hist, inner, (zero, zero))
            ob0[pl.ds(g * LANES, LANES)] = a0
            ob1[pl.ds(g * LANES, LANES)] = a1
            return carry

        lax.fori_loop(0, pool_groups, pool_body, 0)

        osl = pl.ds(wid * batch_per_worker, batch_per_worker)
        pltpu.sync_copy(ob0, out0_hbm.at[osl])
        pltpu.sync_copy(ob1, out1_hbm.at[osl])

    return k(x2, p0, p1)


def kernel(x, emb, W, b):
    batch, hist = x.shape
    p0, p1 = _tc_project(emb.T, W.T, b.reshape(-1, 1), hist)
    bpw = batch // NW
    x2 = (x.astype(jnp.int32)
           .reshape(NW, bpw // LANES, LANES, hist)
           .transpose(0, 1, 3, 2)
           .reshape(NW, bpw * hist))
    out0, out1 = _sc_gather_pool(x2, p0, p1, batch, hist)
    return jnp.stack([out0, out1], axis=1)


# speedup vs baseline: 12.0731x; 1.3049x over previous
import functools

import jax
import jax.numpy as jnp
from jax import lax
from jax.experimental import pallas as pl
from jax.experimental.pallas import tpu as pltpu
from jax.experimental.pallas import tpu_sc as plsc

NC = 2
NS = 16
NW = NC * NS

LANES = 16

VCHUNK = 16384

IDX_PER_STREAM = 128


def _tc_project_pack(embT, Wt, b2, hist):
    E, V = embT.shape
    inv = 1.0 / float(hist)
    grid = (V + VCHUNK - 1) // VCHUNK

    def body(embT_ref, wt_ref, b_ref, o_ref):
        p = jnp.dot(wt_ref[...], embT_ref[...],
                    preferred_element_type=jnp.float32)
        p = (p + b_ref[...]) * inv
        lo = lax.bitcast_convert_type(
            p[0].astype(jnp.bfloat16), jnp.uint16).astype(jnp.uint32)
        hi = lax.bitcast_convert_type(
            p[1].astype(jnp.bfloat16), jnp.uint16).astype(jnp.uint32)
        o_ref[...] = (hi << 16) | lo

    return pl.pallas_call(
        body,
        grid=(grid,),
        in_specs=[
            pl.BlockSpec((E, VCHUNK), lambda i: (0, i)),
            pl.BlockSpec((2, E), lambda i: (0, 0)),
            pl.BlockSpec((2, 1), lambda i: (0, 0)),
        ],
        out_specs=pl.BlockSpec((VCHUNK,), lambda i: (i,)),
        out_shape=jax.ShapeDtypeStruct((V,), jnp.uint32),
    )(embT, Wt, b2)


def _sc_gather_pool(xT, pp, batch, hist):
    batch_per_worker = batch // NW
    rows_per_worker = batch_per_worker * hist
    streams_per_wave = batch_per_worker // IDX_PER_STREAM
    pool_groups = batch_per_worker // LANES
    assert batch_per_worker % IDX_PER_STREAM == 0
    assert batch_per_worker % LANES == 0

    mesh = plsc.VectorSubcoreMesh(
        core_axis_name="c", subcore_axis_name="s",
        num_cores=NC, num_subcores=NS)

    @functools.partial(
        pl.kernel,
        out_type=[
            jax.ShapeDtypeStruct((batch,), jnp.float32),
            jax.ShapeDtypeStruct((batch,), jnp.float32),
        ],
        mesh=mesh,
        scratch_types=[
            pltpu.VMEM((hist, batch_per_worker), jnp.int32),
            pltpu.VMEM((rows_per_worker,), jnp.uint32),
            pltpu.VMEM((batch_per_worker,), jnp.float32),
            pltpu.VMEM((batch_per_worker,), jnp.float32),
            pltpu.SemaphoreType.DMA,
        ],
        compiler_params=pltpu.CompilerParams(
            needs_layout_passes=False, use_tc_tiling_on_sc=False),
    )
    def k(xT_hbm, pp_hbm, out0_hbm, out1_hbm, xv, bv, ob0, ob1, sem):
        wid = lax.axis_index("s") * NC + lax.axis_index("c")
        wslice = pl.ds(wid * batch_per_worker, batch_per_worker)

        pltpu.sync_copy(xT_hbm.at[pl.ds(0, hist), wslice], xv)

        def fire(l):
            for t in range(streams_per_wave):
                pltpu.async_copy(
                    pp_hbm.at[xv.at[l, pl.ds(t * IDX_PER_STREAM,
                                             IDX_PER_STREAM)]],
                    bv.at[pl.ds(l * batch_per_worker + t * IDX_PER_STREAM,
                                IDX_PER_STREAM)],
                    sem)

        def drain_wave():
            pltpu.make_async_copy(
                pp_hbm.at[pl.ds(0, batch_per_worker)],
                bv.at[pl.ds(0, batch_per_worker)], sem).wait()

        fire(0)

        def gather_body(l, carry):
            fire(l)
            drain_wave()
            return carry

        lax.fori_loop(1, hist, gather_body, 0)
        drain_wave()

        def pool_body(g, carry):
            def inner(l, accs):
                a0, a1 = accs
                w = bv[pl.ds(l * batch_per_worker + g * LANES, LANES)]
                e = plsc.bitcast(w << 16, jnp.float32)
                o = plsc.bitcast(w & jnp.uint32(0xFFFF0000), jnp.float32)
                return (a0 + e, a1 + o)

            zero = jnp.zeros((LANES,), jnp.float32)
            a0, a1 = lax.fori_loop(0, hist, inner, (zero, zero))
            ob0[pl.ds(g * LANES, LANES)] = a0
            ob1[pl.ds(g * LANES, LANES)] = a1
            return carry

        lax.fori_loop(0, pool_groups, pool_body, 0)

        pltpu.sync_copy(ob0, out0_hbm.at[wslice])
        pltpu.sync_copy(ob1, out1_hbm.at[wslice])

    return k(xT, pp)


def kernel(x, emb, W, b):
    batch, hist = x.shape
    pp = _tc_project_pack(emb.T, W.T, b.reshape(-1, 1), hist)
    xT = x.astype(jnp.int32).T
    out0, out1 = _sc_gather_pool(xT, pp, batch, hist)
    return jnp.stack([out0, out1], axis=1)


# speedup vs baseline: 14.1495x; 1.1720x over previous
import functools

import jax
import jax.numpy as jnp
from jax import lax
from jax.experimental import pallas as pl
from jax.experimental.pallas import tpu as pltpu
from jax.experimental.pallas import tpu_sc as plsc

NC = 2
NS = 16
NW = NC * NS

LANES = 16

VCHUNK = 32768

IDX_PER_STREAM = 128
WAVES_IN_FLIGHT = 4


def _tc_project_pack(embT, Wt, b2, hist):
    E, V = embT.shape
    inv = 1.0 / float(hist)
    grid = (V + VCHUNK - 1) // VCHUNK

    def body(embT_ref, wt_ref, b_ref, o_ref):
        p = jnp.dot(wt_ref[...], embT_ref[...],
                    preferred_element_type=jnp.float32)
        p = (p + b_ref[...]) * inv
        lo = lax.bitcast_convert_type(
            p[0].astype(jnp.bfloat16), jnp.uint16).astype(jnp.uint32)
        hi = lax.bitcast_convert_type(
            p[1].astype(jnp.bfloat16), jnp.uint16).astype(jnp.uint32)
        o_ref[...] = (hi << 16) | lo

    return pl.pallas_call(
        body,
        grid=(grid,),
        in_specs=[
            pl.BlockSpec((E, VCHUNK), lambda i: (0, i)),
            pl.BlockSpec((2, E), lambda i: (0, 0)),
            pl.BlockSpec((2, 1), lambda i: (0, 0)),
        ],
        out_specs=pl.BlockSpec((VCHUNK,), lambda i: (i,)),
        out_shape=jax.ShapeDtypeStruct((V,), jnp.uint32),
    )(embT, Wt, b2)


def _sc_gather_pool(xT, pp, batch, hist):
    batch_per_worker = batch // NW
    rows_per_worker = batch_per_worker * hist
    streams_per_wave = batch_per_worker // IDX_PER_STREAM
    pool_groups = batch_per_worker // LANES
    assert batch_per_worker % IDX_PER_STREAM == 0
    assert batch_per_worker % LANES == 0

    mesh = plsc.VectorSubcoreMesh(
        core_axis_name="c", subcore_axis_name="s",
        num_cores=NC, num_subcores=NS)

    @functools.partial(
        pl.kernel,
        out_type=[
            jax.ShapeDtypeStruct((batch,), jnp.float32),
            jax.ShapeDtypeStruct((batch,), jnp.float32),
        ],
        mesh=mesh,
        scratch_types=[
            pltpu.VMEM((hist, batch_per_worker), jnp.int32),
            pltpu.VMEM((rows_per_worker,), jnp.uint32),
            pltpu.VMEM((batch_per_worker,), jnp.float32),
            pltpu.VMEM((batch_per_worker,), jnp.float32),
            pltpu.SemaphoreType.DMA,
        ],
        compiler_params=pltpu.CompilerParams(
            needs_layout_passes=False, use_tc_tiling_on_sc=False),
    )
    def k(xT_hbm, pp_hbm, out0_hbm, out1_hbm, xv, bv, ob0, ob1, sem):
        wid = lax.axis_index("s") * NC + lax.axis_index("c")
        wslice = pl.ds(wid * batch_per_worker, batch_per_worker)

        pltpu.sync_copy(xT_hbm.at[pl.ds(0, hist), wslice], xv)

        def fire(l):
            for t in range(streams_per_wave):
                pltpu.async_copy(
                    pp_hbm.at[xv.at[l, pl.ds(t * IDX_PER_STREAM,
                                             IDX_PER_STREAM)]],
                    bv.at[pl.ds(l * batch_per_worker + t * IDX_PER_STREAM,
                                IDX_PER_STREAM)],
                    sem)

        def drain_wave():
            pltpu.make_async_copy(
                pp_hbm.at[pl.ds(0, batch_per_worker)],
                bv.at[pl.ds(0, batch_per_worker)], sem).wait()

        zero = jnp.zeros((LANES,), jnp.float32)

        def zero_body(g, carry):
            ob0[pl.ds(g * LANES, LANES)] = zero
            ob1[pl.ds(g * LANES, LANES)] = zero
            return carry

        lax.fori_loop(0, pool_groups, zero_body, 0)

        for l in range(WAVES_IN_FLIGHT):
            fire(l)

        def gather_body(l, carry):
            @pl.when(l + WAVES_IN_FLIGHT < hist)
            def _():
                fire(l + WAVES_IN_FLIGHT)

            drain_wave()

            def acc_body(g, c):
                sl = pl.ds(g * LANES, LANES)
                w = bv[pl.ds(l * batch_per_worker + g * LANES, LANES)]
                e = plsc.bitcast(w << 16, jnp.float32)
                o = plsc.bitcast(w & jnp.uint32(0xFFFF0000), jnp.float32)
                ob0[sl] = ob0[sl] + e
                ob1[sl] = ob1[sl] + o
                return c

            lax.fori_loop(0, pool_groups, acc_body, 0)
            return carry

        lax.fori_loop(0, hist, gather_body, 0)

        pltpu.sync_copy(ob0, out0_hbm.at[wslice])
        pltpu.sync_copy(ob1, out1_hbm.at[wslice])

    return k(xT, pp)


def kernel(x, emb, W, b):
    batch, hist = x.shape
    pp = _tc_project_pack(emb.T, W.T, b.reshape(-1, 1), hist)
    xT = x.astype(jnp.int32).T
    out0, out1 = _sc_gather_pool(xT, pp, batch, hist)
    return jnp.stack([out0, out1], axis=1)


# speedup vs baseline: 14.3225x; 1.0122x over previous
import functools

import jax
import jax.numpy as jnp
from jax import lax
from jax.experimental import pallas as pl
from jax.experimental.pallas import tpu as pltpu
from jax.experimental.pallas import tpu_sc as plsc

NC = 2
NS = 16
NW = NC * NS

LANES = 16

VCHUNK = 65536

IDX_PER_STREAM = 128
WAVES_IN_FLIGHT = 8


def _tc_project_pack(embT, Wt, b2, hist):
    E, V = embT.shape
    inv = 1.0 / float(hist)
    grid = (V + VCHUNK - 1) // VCHUNK

    def body(embT_ref, wt_ref, b_ref, o_ref):
        p = jnp.dot(wt_ref[...], embT_ref[...],
                    preferred_element_type=jnp.float32)
        p = (p + b_ref[...]) * inv
        lo = lax.bitcast_convert_type(
            p[0].astype(jnp.bfloat16), jnp.uint16).astype(jnp.uint32)
        hi = lax.bitcast_convert_type(
            p[1].astype(jnp.bfloat16), jnp.uint16).astype(jnp.uint32)
        o_ref[...] = (hi << 16) | lo

    return pl.pallas_call(
        body,
        grid=(grid,),
        in_specs=[
            pl.BlockSpec((E, VCHUNK), lambda i: (0, i)),
            pl.BlockSpec((2, E), lambda i: (0, 0)),
            pl.BlockSpec((2, 1), lambda i: (0, 0)),
        ],
        out_specs=pl.BlockSpec((VCHUNK,), lambda i: (i,)),
        out_shape=jax.ShapeDtypeStruct((V,), jnp.uint32),
    )(embT, Wt, b2)


def _sc_gather_pool(xT, pp, batch, hist):
    batch_per_worker = batch // NW
    rows_per_worker = batch_per_worker * hist
    streams_per_wave = batch_per_worker // IDX_PER_STREAM
    pool_groups = batch_per_worker // LANES
    assert batch_per_worker % IDX_PER_STREAM == 0
    assert batch_per_worker % LANES == 0

    mesh = plsc.VectorSubcoreMesh(
        core_axis_name="c", subcore_axis_name="s",
        num_cores=NC, num_subcores=NS)

    @functools.partial(
        pl.kernel,
        out_type=[
            jax.ShapeDtypeStruct((batch,), jnp.float32),
            jax.ShapeDtypeStruct((batch,), jnp.float32),
        ],
        mesh=mesh,
        scratch_types=[
            pltpu.VMEM((hist, batch_per_worker), jnp.int32),
            pltpu.VMEM((rows_per_worker,), jnp.uint32),
            pltpu.VMEM((batch_per_worker,), jnp.float32),
            pltpu.VMEM((batch_per_worker,), jnp.float32),
            pltpu.SemaphoreType.DMA,
        ],
        compiler_params=pltpu.CompilerParams(
            needs_layout_passes=False, use_tc_tiling_on_sc=False),
    )
    def k(xT_hbm, pp_hbm, out0_hbm, out1_hbm, xv, bv, ob0, ob1, sem):
        wid = lax.axis_index("s") * NC + lax.axis_index("c")
        wslice = pl.ds(wid * batch_per_worker, batch_per_worker)

        pltpu.sync_copy(xT_hbm.at[pl.ds(0, hist), wslice], xv)

        def fire(l):
            for t in range(streams_per_wave):
                pltpu.async_copy(
                    pp_hbm.at[xv.at[l, pl.ds(t * IDX_PER_STREAM,
                                             IDX_PER_STREAM)]],
                    bv.at[pl.ds(l * batch_per_worker + t * IDX_PER_STREAM,
                                IDX_PER_STREAM)],
                    sem)

        def drain_wave():
            pltpu.make_async_copy(
                pp_hbm.at[pl.ds(0, batch_per_worker)],
                bv.at[pl.ds(0, batch_per_worker)], sem).wait()

        zero = jnp.zeros((LANES,), jnp.float32)

        def zero_body(g, carry):
            ob0[pl.ds(g * LANES, LANES)] = zero
            ob1[pl.ds(g * LANES, LANES)] = zero
            return carry

        lax.fori_loop(0, pool_groups, zero_body, 0)

        for l in range(WAVES_IN_FLIGHT):
            fire(l)

        def gather_body(l, carry):
            @pl.when(l + WAVES_IN_FLIGHT < hist)
            def _():
                fire(l + WAVES_IN_FLIGHT)

            drain_wave()

            def acc_body(g, c):
                sl = pl.ds(g * LANES, LANES)
                w = bv[pl.ds(l * batch_per_worker + g * LANES, LANES)]
                e = plsc.bitcast(w << 16, jnp.float32)
                o = plsc.bitcast(w & jnp.uint32(0xFFFF0000), jnp.float32)
                ob0[sl] = ob0[sl] + e
                ob1[sl] = ob1[sl] + o
                return c

            lax.fori_loop(0, pool_groups, acc_body, 0)
            return carry

        lax.fori_loop(0, hist, gather_body, 0)

        pltpu.sync_copy(ob0, out0_hbm.at[wslice])
        pltpu.sync_copy(ob1, out1_hbm.at[wslice])

    return k(xT, pp)


def kernel(x, emb, W, b):
    batch, hist = x.shape
    pp = _tc_project_pack(emb.T, W.T, b.reshape(-1, 1), hist)
    xT = x.astype(jnp.int32).T
    out0, out1 = _sc_gather_pool(xT, pp, batch, hist)
    return jnp.stack([out0, out1], axis=1)
